# baseline (speedup 1.0000x reference)
import functools

import jax
import jax.numpy as jnp
from jax import lax
from jax.experimental import pallas as pl
from jax.experimental.pallas import tpu as pltpu
from jax.experimental.pallas import tpu_sc as plsc

_B, _Q, _C, _T = 8, 900, 91, 25
_ALPHA = 0.25
_N = _B * _T
_CHUNK_OFFS = tuple(range(0, 192, 16)) + (184,)


def _ce_body(logits_ref, src_col_ref, src_row_ref, lab_col_ref, out_ref):
    b = pl.program_id(0)
    x = logits_ref[0]
    src_c = src_col_ref[0]
    src_r = src_row_ref[0]
    lab_c = lab_col_ref[0]

    e = jnp.exp(-jnp.abs(x))
    onep = 1.0 + e
    sp = jnp.maximum(x, 0.0) + jnp.log1p(e)
    sig = jnp.where(x >= 0.0, 1.0, e) / onep
    neg = (1.0 - _ALPHA) * sig * sig * sp
    partial = jnp.sum(neg, axis=0, keepdims=True)

    ti = lax.broadcasted_iota(jnp.int32, (_T, _T), 0)
    tj = lax.broadcasted_iota(jnp.int32, (_T, _T), 1)
    dup_later = (src_c == src_r) & (tj > ti)
    wins_c = jnp.logical_not(jnp.any(dup_later, axis=1, keepdims=True))

    qi = lax.broadcasted_iota(jnp.int32, (_T, _Q), 1)
    m1 = jnp.where(qi == src_c, 1.0, 0.0)
    g = jnp.dot(m1, x, preferred_element_type=jnp.float32)
    ohl = jnp.where(
        lax.broadcasted_iota(jnp.int32, (_T, _C), 1) == lab_c, 1.0, 0.0
    )
    xg = jnp.sum(g * ohl, axis=1, keepdims=True)

    eg = jnp.exp(-jnp.abs(xg))
    onepg = 1.0 + eg
    spg = jnp.maximum(xg, 0.0) + jnp.log1p(eg)
    sigg = jnp.where(xg >= 0.0, 1.0, eg) / onepg
    negg = (1.0 - _ALPHA) * sigg * sigg * spg
    posg = _ALPHA * (1.0 - sigg) * (1.0 - sigg) * (spg - xg)
    d = jnp.where(wins_c, posg - negg, 0.0)
    corr = jnp.sum(ohl * d, axis=0, keepdims=True)

    @pl.when(b == 0)
    def _init():
        out_ref[...] = jnp.zeros((1, _C), jnp.float32)

    out_ref[...] += partial + corr


def _ce_call(pred_logits, src_i, lab_i):
    return pl.pallas_call(
        _ce_body,
        grid=(_B,),
        in_specs=[
            pl.BlockSpec((1, _Q, _C), lambda b: (b, 0, 0)),
            pl.BlockSpec((1, _T, 1), lambda b: (b, 0, 0)),
            pl.BlockSpec((1, 1, _T), lambda b: (b, 0, 0)),
            pl.BlockSpec((1, _T, 1), lambda b: (b, 0, 0)),
        ],
        out_specs=pl.BlockSpec((1, _C), lambda b: (0, 0)),
        out_shape=jax.ShapeDtypeStruct((1, _C), jnp.float32),
    )(pred_logits, src_i[:, :, None], src_i[:, None, :], lab_i[:, :, None])


def _box_body(pred_hbm, tgt_hbm, src_hbm, out_hbm, pred_v, tgt_v, idx_v, out_v, sem):
    wid = lax.axis_index("s") * 2 + lax.axis_index("c")

    @pl.when(wid == 0)
    def _():
        cp1 = pltpu.async_copy(pred_hbm, pred_v, sem)
        cp2 = pltpu.async_copy(tgt_hbm, tgt_v, sem)
        cp3 = pltpu.async_copy(src_hbm, idx_v, sem)
        cp1.wait()
        cp2.wait()
        cp3.wait()
        iot = lax.broadcasted_iota(jnp.int32, (16,), 0)
        l1_acc = jnp.zeros((16,), jnp.float32)
        gi_acc = jnp.zeros((16,), jnp.float32)
        for off in _CHUNK_OFFS:
            t_vec = iot + off
            b0 = off // _T
            bq = b0 * _Q + jnp.where(t_vec >= (b0 + 1) * _T, _Q, 0)
            rows = idx_v[pl.ds(off, 16)] + bq
            sof = rows * 4
            tof = t_vec * 4

            def _g(ref, base, c):
                return plsc.load_gather(ref, [base + c])

            scx = _g(pred_v, sof, 0)
            scy = _g(pred_v, sof, 1)
            sw = _g(pred_v, sof, 2)
            sh = _g(pred_v, sof, 3)
            tcx = _g(tgt_v, tof, 0)
            tcy = _g(tgt_v, tof, 1)
            tw = _g(tgt_v, tof, 2)
            th = _g(tgt_v, tof, 3)

            l1 = (jnp.abs(scx - tcx) + jnp.abs(scy - tcy)
                  + jnp.abs(sw - tw) + jnp.abs(sh - th))

            sx0 = scx - 0.5 * sw
            sy0 = scy - 0.5 * sh
            sx1 = scx + 0.5 * sw
            sy1 = scy + 0.5 * sh
            tx0 = tcx - 0.5 * tw
            ty0 = tcy - 0.5 * th
            tx1 = tcx + 0.5 * tw
            ty1 = tcy + 0.5 * th

            area1 = (sx1 - sx0) * (sy1 - sy0)
            area2 = (tx1 - tx0) * (ty1 - ty0)
            wi = jnp.maximum(jnp.minimum(sx1, tx1) - jnp.maximum(sx0, tx0), 0.0)
            hi = jnp.maximum(jnp.minimum(sy1, ty1) - jnp.maximum(sy0, ty0), 0.0)
            inter = wi * hi
            union = area1 + area2 - inter
            iou = inter / union
            we = jnp.maximum(jnp.maximum(sx1, tx1) - jnp.minimum(sx0, tx0), 0.0)
            he = jnp.maximum(jnp.maximum(sy1, ty1) - jnp.minimum(sy0, ty0), 0.0)
            areae = we * he
            giou = iou - (areae - union) / areae

            one_m_giou = 1.0 - giou
            if off == _CHUNK_OFFS[-1]:
                fresh = iot >= 8
                l1 = jnp.where(fresh, l1, 0.0)
                one_m_giou = jnp.where(fresh, one_m_giou, 0.0)
            l1_acc = l1_acc + l1
            gi_acc = gi_acc + one_m_giou
        out_v[pl.ds(0, 16)] = l1_acc
        out_v[pl.ds(16, 16)] = gi_acc
        pltpu.sync_copy(out_v, out_hbm)


@functools.cache
def _get_box_call():
    mesh = plsc.VectorSubcoreMesh(core_axis_name="c", subcore_axis_name="s")
    return pl.kernel(
        _box_body,
        mesh=mesh,
        compiler_params=pltpu.CompilerParams(needs_layout_passes=False),
        out_type=jax.ShapeDtypeStruct((32,), jnp.float32),
        scratch_types=[
            pltpu.VMEM((_B * _Q * 4,), jnp.float32),
            pltpu.VMEM((_N * 4,), jnp.float32),
            pltpu.VMEM((_N,), jnp.int32),
            pltpu.VMEM((32,), jnp.float32),
            pltpu.SemaphoreType.DMA,
        ],
    )


def kernel(pred_logits, pred_boxes, tgt_boxes, tgt_labels, src_idx):
    src_i = src_idx.astype(jnp.int32)
    lab_i = tgt_labels.astype(jnp.int32)

    ce = _ce_call(pred_logits, src_i, lab_i)
    box = _get_box_call()(
        pred_boxes.reshape(-1), tgt_boxes.reshape(-1), src_i.reshape(-1)
    )

    nb = jnp.float32(_N)
    return jnp.stack([
        jnp.sum(ce) / nb,
        jnp.sum(box[:16]) / nb,
        jnp.sum(box[16:]) / nb,
    ])


# speedup vs baseline: 1.4891x; 1.4891x over previous
import functools

import jax
import jax.numpy as jnp
from jax import lax
from jax.experimental import pallas as pl
from jax.experimental.pallas import tpu as pltpu
from jax.experimental.pallas import tpu_sc as plsc

_B, _Q, _C, _T = 8, 900, 91, 25
_ALPHA = 0.25
_N = _B * _T
_CB = 13
_CHUNK_OFFS = tuple(range(0, 192, 16)) + (184,)


def _ce_body(logits_ref, src_ref, lab_ref, out_ref, tcq_ref):
    i = pl.program_id(0)

    @pl.when(i == 0)
    def _build_map():
        qi = lax.broadcasted_iota(jnp.int32, (_B, _Q), 1)
        tcq = jnp.full((_B, _Q), -1, jnp.int32)
        for t in range(_T):
            m = qi == src_ref[:, t:t + 1]
            tcq = jnp.where(m, lab_ref[:, t:t + 1], tcq)
        tcq_ref[...] = tcq
        out_ref[...] = jnp.zeros((1, 128), jnp.float32)

    x = logits_ref[...]
    ci = lax.broadcasted_iota(jnp.int32, (_CB, _B, _Q), 0) + i * _CB
    ispos = ci == tcq_ref[...][None, :, :]

    e = jnp.exp(-jnp.abs(x))
    onep = 1.0 + e
    sp = jnp.maximum(x, 0.0) + jnp.log1p(e)
    sig = jnp.where(x >= 0.0, 1.0, e) / onep
    neg = (1.0 - _ALPHA) * sig * sig * sp
    pos = _ALPHA * (1.0 - sig) * (1.0 - sig) * (sp - x)
    total = jnp.sum(jnp.where(ispos, pos, neg))

    out_ref[...] += total + jnp.zeros((1, 128), jnp.float32)


def _ce_call(logits_t, src_i, lab_i):
    return pl.pallas_call(
        _ce_body,
        grid=(_C // _CB,),
        in_specs=[
            pl.BlockSpec((_CB, _B, _Q), lambda i: (i, 0, 0)),
            pl.BlockSpec((_B, _T), lambda i: (0, 0)),
            pl.BlockSpec((_B, _T), lambda i: (0, 0)),
        ],
        out_specs=pl.BlockSpec((1, 128), lambda i: (0, 0)),
        out_shape=jax.ShapeDtypeStruct((1, 128), jnp.float32),
        scratch_shapes=[pltpu.VMEM((_B, _Q), jnp.int32)],
    )(logits_t, src_i, lab_i)


def _box_body(pred_hbm, tgt_hbm, src_hbm, out_hbm, pred_v, tgt_v, idx_v, out_v, sem):
    wid = lax.axis_index("s") * 2 + lax.axis_index("c")

    @pl.when(wid == 0)
    def _():
        cp1 = pltpu.async_copy(pred_hbm, pred_v, sem)
        cp2 = pltpu.async_copy(tgt_hbm, tgt_v, sem)
        cp3 = pltpu.async_copy(src_hbm, idx_v, sem)
        cp1.wait()
        cp2.wait()
        cp3.wait()
        iot = lax.broadcasted_iota(jnp.int32, (16,), 0)
        l1_acc = jnp.zeros((16,), jnp.float32)
        gi_acc = jnp.zeros((16,), jnp.float32)
        for off in _CHUNK_OFFS:
            t_vec = iot + off
            b0 = off // _T
            b_vec = b0 + jnp.where(t_vec >= (b0 + 1) * _T, 1, 0)
            tt_vec = t_vec - b_vec * _T
            q_vec = plsc.load_gather(idx_v, [b_vec, tt_vec])

            def _g(ref, c, col):
                cc = jnp.full((16,), c, jnp.int32)
                return plsc.load_gather(ref, [b_vec, cc, col])

            scx = _g(pred_v, 0, q_vec)
            scy = _g(pred_v, 1, q_vec)
            sw = _g(pred_v, 2, q_vec)
            sh = _g(pred_v, 3, q_vec)
            tcx = _g(tgt_v, 0, tt_vec)
            tcy = _g(tgt_v, 1, tt_vec)
            tw = _g(tgt_v, 2, tt_vec)
            th = _g(tgt_v, 3, tt_vec)

            l1 = (jnp.abs(scx - tcx) + jnp.abs(scy - tcy)
                  + jnp.abs(sw - tw) + jnp.abs(sh - th))

            sx0 = scx - 0.5 * sw
            sy0 = scy - 0.5 * sh
            sx1 = scx + 0.5 * sw
            sy1 = scy + 0.5 * sh
            tx0 = tcx - 0.5 * tw
            ty0 = tcy - 0.5 * th
            tx1 = tcx + 0.5 * tw
            ty1 = tcy + 0.5 * th

            area1 = (sx1 - sx0) * (sy1 - sy0)
            area2 = (tx1 - tx0) * (ty1 - ty0)
            wi = jnp.maximum(jnp.minimum(sx1, tx1) - jnp.maximum(sx0, tx0), 0.0)
            hi = jnp.maximum(jnp.minimum(sy1, ty1) - jnp.maximum(sy0, ty0), 0.0)
            inter = wi * hi
            union = area1 + area2 - inter
            iou = inter / union
            we = jnp.maximum(jnp.maximum(sx1, tx1) - jnp.minimum(sx0, tx0), 0.0)
            he = jnp.maximum(jnp.maximum(sy1, ty1) - jnp.minimum(sy0, ty0), 0.0)
            areae = we * he
            giou = iou - (areae - union) / areae

            one_m_giou = 1.0 - giou
            if off == _CHUNK_OFFS[-1]:
                fresh = iot >= 8
                l1 = jnp.where(fresh, l1, 0.0)
                one_m_giou = jnp.where(fresh, one_m_giou, 0.0)
            l1_acc = l1_acc + l1
            gi_acc = gi_acc + one_m_giou
        out_v[pl.ds(0, 16)] = l1_acc
        out_v[pl.ds(16, 16)] = gi_acc
        pltpu.sync_copy(out_v, out_hbm)


@functools.cache
def _get_box_call():
    mesh = plsc.VectorSubcoreMesh(core_axis_name="c", subcore_axis_name="s")
    return pl.kernel(
        _box_body,
        mesh=mesh,
        compiler_params=pltpu.CompilerParams(needs_layout_passes=False),
        out_type=jax.ShapeDtypeStruct((32,), jnp.float32),
        scratch_types=[
            pltpu.VMEM((_B, 4, _Q), jnp.float32),
            pltpu.VMEM((_B, 4, _T), jnp.float32),
            pltpu.VMEM((_B, _T), jnp.int32),
            pltpu.VMEM((32,), jnp.float32),
            pltpu.SemaphoreType.DMA,
        ],
    )


def kernel(pred_logits, pred_boxes, tgt_boxes, tgt_labels, src_idx):
    src_i = src_idx.astype(jnp.int32)
    lab_i = tgt_labels.astype(jnp.int32)

    ce = _ce_call(jnp.transpose(pred_logits, (2, 0, 1)), src_i, lab_i)
    box = _get_box_call()(
        jnp.transpose(pred_boxes, (0, 2, 1)),
        jnp.transpose(tgt_boxes, (0, 2, 1)),
        src_i,
    )

    nb = jnp.float32(_N)
    return jnp.stack([
        ce[0, 0] / nb,
        jnp.sum(box[:16]) / nb,
        jnp.sum(box[16:]) / nb,
    ])


# speedup vs baseline: 1.6778x; 1.1267x over previous
import functools

import jax
import jax.numpy as jnp
from jax import lax
from jax.experimental import pallas as pl
from jax.experimental.pallas import tpu as pltpu
from jax.experimental.pallas import tpu_sc as plsc

_B, _Q, _C, _T = 8, 900, 91, 25
_ALPHA = 0.25
_N = _B * _T
_CB = 13
_CHUNK_OFFS = tuple(range(0, 192, 16)) + (184,)


def _ce_body(logits_ref, src_ref, lab_ref, out_ref, tcq_ref):
    i = pl.program_id(0)

    @pl.when(i == 0)
    def _build_map():
        qi = lax.broadcasted_iota(jnp.int32, (_B, _Q), 1)
        tcq = jnp.full((_B, _Q), -1, jnp.int32)
        for t in range(_T):
            m = qi == src_ref[:, t:t + 1]
            tcq = jnp.where(m, lab_ref[:, t:t + 1], tcq)
        tcq_ref[...] = tcq
        out_ref[...] = jnp.zeros((1, 128), jnp.float32)

    tcq = tcq_ref[...]
    base = i * _CB
    acc = jnp.zeros((_B, _Q), jnp.float32)
    for c in range(_CB):
        x = logits_ref[c]
        ispos = tcq == base + c
        e = jnp.exp(-jnp.abs(x))
        onep = 1.0 + e
        sp = jnp.maximum(x, 0.0) + jnp.log(onep)
        v = jnp.where(ispos == (x >= 0.0), e, 1.0)
        k = jnp.where(ispos, 0.5, 0.8660254037844386)
        vk = v * k
        spx = sp - jnp.where(ispos, x, 0.0)
        u = 1.0 / onep
        acc = acc + (vk * vk) * (spx * (u * u))

    out_ref[...] += jnp.sum(acc) + jnp.zeros((1, 128), jnp.float32)


def _ce_call(logits_t, src_i, lab_i):
    return pl.pallas_call(
        _ce_body,
        grid=(_C // _CB,),
        in_specs=[
            pl.BlockSpec((_CB, _B, _Q), lambda i: (i, 0, 0)),
            pl.BlockSpec((_B, _T), lambda i: (0, 0)),
            pl.BlockSpec((_B, _T), lambda i: (0, 0)),
        ],
        out_specs=pl.BlockSpec((1, 128), lambda i: (0, 0)),
        out_shape=jax.ShapeDtypeStruct((1, 128), jnp.float32),
        scratch_shapes=[pltpu.VMEM((_B, _Q), jnp.int32)],
    )(logits_t, src_i, lab_i)


def _box_body(pred_hbm, tgt_hbm, src_hbm, out_hbm, pred_v, tgt_v, idx_v, out_v, sem):
    wid = lax.axis_index("s") * 2 + lax.axis_index("c")

    @pl.when(wid == 0)
    def _():
        cp1 = pltpu.async_copy(pred_hbm, pred_v, sem)
        cp2 = pltpu.async_copy(tgt_hbm, tgt_v, sem)
        cp3 = pltpu.async_copy(src_hbm, idx_v, sem)
        cp1.wait()
        cp2.wait()
        cp3.wait()
        iot = lax.broadcasted_iota(jnp.int32, (16,), 0)
        l1_acc = jnp.zeros((16,), jnp.float32)
        gi_acc = jnp.zeros((16,), jnp.float32)
        for off in _CHUNK_OFFS:
            t_vec = iot + off
            b0 = off // _T
            b_vec = b0 + jnp.where(t_vec >= (b0 + 1) * _T, 1, 0)
            tt_vec = t_vec - b_vec * _T
            q_vec = plsc.load_gather(idx_v, [b_vec, tt_vec])

            def _g(ref, c, col):
                cc = jnp.full((16,), c, jnp.int32)
                return plsc.load_gather(ref, [b_vec, cc, col])

            scx = _g(pred_v, 0, q_vec)
            scy = _g(pred_v, 1, q_vec)
            sw = _g(pred_v, 2, q_vec)
            sh = _g(pred_v, 3, q_vec)
            tcx = _g(tgt_v, 0, tt_vec)
            tcy = _g(tgt_v, 1, tt_vec)
            tw = _g(tgt_v, 2, tt_vec)
            th = _g(tgt_v, 3, tt_vec)

            l1 = (jnp.abs(scx - tcx) + jnp.abs(scy - tcy)
                  + jnp.abs(sw - tw) + jnp.abs(sh - th))

            sx0 = scx - 0.5 * sw
            sy0 = scy - 0.5 * sh
            sx1 = scx + 0.5 * sw
            sy1 = scy + 0.5 * sh
            tx0 = tcx - 0.5 * tw
            ty0 = tcy - 0.5 * th
            tx1 = tcx + 0.5 * tw
            ty1 = tcy + 0.5 * th

            area1 = (sx1 - sx0) * (sy1 - sy0)
            area2 = (tx1 - tx0) * (ty1 - ty0)
            wi = jnp.maximum(jnp.minimum(sx1, tx1) - jnp.maximum(sx0, tx0), 0.0)
            hi = jnp.maximum(jnp.minimum(sy1, ty1) - jnp.maximum(sy0, ty0), 0.0)
            inter = wi * hi
            union = area1 + area2 - inter
            iou = inter / union
            we = jnp.maximum(jnp.maximum(sx1, tx1) - jnp.minimum(sx0, tx0), 0.0)
            he = jnp.maximum(jnp.maximum(sy1, ty1) - jnp.minimum(sy0, ty0), 0.0)
            areae = we * he
            giou = iou - (areae - union) / areae

            one_m_giou = 1.0 - giou
            if off == _CHUNK_OFFS[-1]:
                fresh = iot >= 8
                l1 = jnp.where(fresh, l1, 0.0)
                one_m_giou = jnp.where(fresh, one_m_giou, 0.0)
            l1_acc = l1_acc + l1
            gi_acc = gi_acc + one_m_giou
        inv_nb = 1.0 / jnp.float32(_N)
        l1_s = jnp.sum(l1_acc) * inv_nb
        gi_s = jnp.sum(gi_acc) * inv_nb
        out_v[...] = jnp.where(iot == 0, l1_s, jnp.where(iot == 1, gi_s, 0.0))
        pltpu.sync_copy(out_v, out_hbm)


@functools.cache
def _get_box_call():
    mesh = plsc.VectorSubcoreMesh(core_axis_name="c", subcore_axis_name="s")
    return pl.kernel(
        _box_body,
        mesh=mesh,
        compiler_params=pltpu.CompilerParams(needs_layout_passes=False),
        out_type=jax.ShapeDtypeStruct((16,), jnp.float32),
        scratch_types=[
            pltpu.VMEM((_B, 4, _Q), jnp.float32),
            pltpu.VMEM((_B, 4, _T), jnp.float32),
            pltpu.VMEM((_B, _T), jnp.int32),
            pltpu.VMEM((16,), jnp.float32),
            pltpu.SemaphoreType.DMA,
        ],
    )


def kernel(pred_logits, pred_boxes, tgt_boxes, tgt_labels, src_idx):
    src_i = src_idx.astype(jnp.int32)
    lab_i = tgt_labels.astype(jnp.int32)

    ce = _ce_call(jnp.transpose(pred_logits, (2, 0, 1)), src_i, lab_i)
    box = _get_box_call()(
        jnp.transpose(pred_boxes, (0, 2, 1)),
        jnp.transpose(tgt_boxes, (0, 2, 1)),
        src_i,
    )

    return jnp.stack([ce[0, 0] / jnp.float32(_N), box[0], box[1]])
